# Initial kernel scaffold; baseline (speedup 1.0000x reference)
import jax
import jax.numpy as jnp
from jax.experimental import pallas as pl


def kernel(roi_bboxes, gt_boxes, gt_labels, gt_box_indices):
    raise NotImplementedError("write your pallas kernel here")


# speedup vs baseline: 7.0932x; 7.0932x over previous
import functools

import jax
import jax.numpy as jnp
from jax import lax
from jax.experimental import pallas as pl
from jax.experimental.pallas import tpu as pltpu
from jax.experimental.pallas import tpu_sc as plsc

B = 64
TOTAL = 256
POS = 64
NEG = TOTAL - POS
NLAB = 81
DROW = 4 * NLAB
MAXGT = 100
GTL_PAD = 104
NC, NS = 2, 16
NW = NC * NS
BPW = B // NW

DSIZE = TOTAL * DROW
LSIZE = TOTAL * NLAB

_LN2 = 0.6931471805599453
_SQRT2 = 1.4142135623730951


def _vlog(x):
    bits = lax.bitcast_convert_type(x, jnp.int32)
    e = (bits >> 23) - 127
    m = lax.bitcast_convert_type((bits & 0x7FFFFF) | 0x3F800000, jnp.float32)
    big = m > _SQRT2
    m = jnp.where(big, m * 0.5, m)
    e = e + big.astype(jnp.int32)
    s = (m - 1.0) / (m + 1.0)
    s2 = s * s
    p = 2.0 * s * (1.0 + s2 * (1.0 / 3.0 + s2 * (0.2 + s2 * (1.0 / 7.0))))
    return e.astype(jnp.float32) * _LN2 + p


def _sc_body(roi_h, gtb_h, gtl_h, gti_h, d_out, l_out,
             dbuf, lbuf, roi_v, gtb_v, gtl_v, idx_v, offs_v):
    wid = lax.axis_index("s") * NC + lax.axis_index("c")
    iota = lax.iota(jnp.int32, 16)
    zf = jnp.zeros((16,), jnp.float32)
    zi = jnp.zeros((16,), jnp.int32)
    ones = jnp.ones((16,), jnp.int32)

    def _zero_d(i, c):
        base = i * 256
        for j in range(16):
            dbuf[pl.ds(base + j * 16, 16)] = zf
        return c

    lax.fori_loop(0, DSIZE // 256, _zero_d, 0)

    def _zero_l(i, c):
        base = i * 256
        for j in range(16):
            lbuf[pl.ds(base + j * 16, 16)] = zi
        return c

    lax.fori_loop(0, LSIZE // 256, _zero_l, 0)

    for c in range(NEG // 16):
        t = POS + c * 16 + iota
        plsc.store_scatter(lbuf, [t * NLAB + (NLAB - 1)], ones)

    for bb in range(BPW):
        b = wid * BPW + bb
        pltpu.sync_copy(roi_h.at[b], roi_v)
        pltpu.sync_copy(gtb_h.at[b], gtb_v)
        pltpu.sync_copy(gtl_h.at[b], gtl_v)
        pltpu.sync_copy(gti_h.at[b], idx_v)
        if bb > 0:
            for c in range(POS // 16):
                od = offs_v[pl.ds(c * 16, 16)]
                for k in range(4):
                    plsc.store_scatter(dbuf, [od + k], zf)
                ol = offs_v[pl.ds(64 + c * 16, 16)]
                plsc.store_scatter(lbuf, [ol], zi)
        for c in range(POS // 16):
            gi = idx_v[pl.ds(c * 16, 16)]
            gb = gi * 4
            gy1 = plsc.load_gather(gtb_v, [gb])
            gx1 = plsc.load_gather(gtb_v, [gb + 1])
            gy2 = plsc.load_gather(gtb_v, [gb + 2])
            gx2 = plsc.load_gather(gtb_v, [gb + 3])
            lg = plsc.load_gather(gtl_v, [gi])
            rb = iota * 4 + c * 64
            ry1 = plsc.load_gather(roi_v, [rb])
            rx1 = plsc.load_gather(roi_v, [rb + 1])
            ry2 = plsc.load_gather(roi_v, [rb + 2])
            rx2 = plsc.load_gather(roi_v, [rb + 3])
            bw = rx2 - rx1
            bh = ry2 - ry1
            bcx = rx1 + 0.5 * bw
            bcy = ry1 + 0.5 * bh
            gw = gx2 - gx1
            gh = gy2 - gy1
            gcx = gx1 + 0.5 * gw
            gcy = gy1 + 0.5 * gh
            bw = jnp.where(bw == 0.0, 1e-3, bw)
            bh = jnp.where(bh == 0.0, 1e-3, bh)
            wz = gw == 0.0
            hz = gh == 0.0
            dx = jnp.where(wz, 0.0, (gcx - bcx) / bw)
            dy = jnp.where(hz, 0.0, (gcy - bcy) / bh)
            dw = jnp.where(wz, 0.0, _vlog(gw / bw))
            dh = jnp.where(hz, 0.0, _vlog(gh / bh))
            t = c * 16 + iota
            od = t * DROW + lg * 4
            ol = t * NLAB + lg
            plsc.store_scatter(dbuf, [od], dy)
            plsc.store_scatter(dbuf, [od + 1], dx)
            plsc.store_scatter(dbuf, [od + 2], dh)
            plsc.store_scatter(dbuf, [od + 3], dw)
            plsc.store_scatter(lbuf, [ol], ones)
            if bb + 1 < BPW:
                offs_v[pl.ds(c * 16, 16)] = od
                offs_v[pl.ds(64 + c * 16, 16)] = ol
        pltpu.sync_copy(dbuf, d_out.at[b])
        pltpu.sync_copy(lbuf, l_out.at[b])


_sc_call = functools.partial(
    pl.kernel,
    out_type=[
        jax.ShapeDtypeStruct((B, DSIZE), jnp.float32),
        jax.ShapeDtypeStruct((B, LSIZE), jnp.int32),
    ],
    mesh=plsc.VectorSubcoreMesh(core_axis_name="c", subcore_axis_name="s",
                                num_cores=NC),
    compiler_params=pltpu.CompilerParams(needs_layout_passes=False),
    scratch_types=[
        pltpu.VMEM((DSIZE,), jnp.float32),
        pltpu.VMEM((LSIZE,), jnp.int32),
        pltpu.VMEM((TOTAL * 4,), jnp.float32),
        pltpu.VMEM((MAXGT * 4,), jnp.float32),
        pltpu.VMEM((GTL_PAD,), jnp.int32),
        pltpu.VMEM((POS,), jnp.int32),
        pltpu.VMEM((128,), jnp.int32),
    ],
)(_sc_body)


def kernel(roi_bboxes, gt_boxes, gt_labels, gt_box_indices):
    roi_f = roi_bboxes.reshape(B, TOTAL * 4)
    gtb_f = gt_boxes.reshape(B, MAXGT * 4)
    gtl_p = jnp.pad(gt_labels, ((0, 0), (0, GTL_PAD - MAXGT)))
    d_out, l_out = _sc_call(roi_f, gtb_f, gtl_p, gt_box_indices)
    return (d_out.reshape(B, TOTAL, DROW), l_out.reshape(B, TOTAL, NLAB))
